# baseline (speedup 1.0000x reference)
import functools

import jax
import jax.numpy as jnp
from jax import lax
from jax.experimental import pallas as pl
from jax.experimental.pallas import tpu as pltpu
from jax.experimental.pallas import tpu_sc as plsc

N_NODES = 10000
N_EDGES = 320000
IN_C = 128
OUT_C = 128
EIN = 16
EOUT = 16
EPS = 1e-5

NC = 2
NS = 16
NW = NC * NS
EPW = N_EDGES // NW
CHUNK = 80
NCHUNK = EPW // CHUNK
NPAIR = NCHUNK // 2
TILE_ROWS = 624
TAIL_ROWS = N_NODES - NS * TILE_ROWS


def _sc_mesh():
    return plsc.VectorSubcoreMesh(
        core_axis_name="c", subcore_axis_name="s", num_cores=NC, num_subcores=NS
    )


def _gather_sum_packed(row3, col3, p32, q32, ne, chunk):
    epw = ne // NW
    nchunk = epw // chunk
    npair = nchunk // 2
    dw = OUT_C // 2

    @functools.partial(
        pl.kernel,
        out_type=jax.ShapeDtypeStruct((ne, OUT_C), jnp.float32),
        mesh=_sc_mesh(),
        scratch_types=[
            pltpu.VMEM((nchunk, chunk), jnp.int32),
            pltpu.VMEM((nchunk, chunk), jnp.int32),
            pltpu.VMEM((chunk, dw), jnp.int32),
            pltpu.VMEM((chunk, dw), jnp.int32),
            pltpu.VMEM((chunk, dw), jnp.int32),
            pltpu.VMEM((chunk, dw), jnp.int32),
            pltpu.VMEM((chunk, OUT_C), jnp.float32),
            pltpu.VMEM((chunk, OUT_C), jnp.float32),
            pltpu.SemaphoreType.DMA,
            pltpu.SemaphoreType.DMA,
            pltpu.SemaphoreType.DMA,
            pltpu.SemaphoreType.DMA,
            pltpu.SemaphoreType.DMA,
        ],
        compiler_params=pltpu.CompilerParams(
            use_tc_tiling_on_sc=False, needs_layout_passes=False
        ),
    )
    def k(row_hbm, col_hbm, p_hbm, q_hbm, g_hbm,
          idxr, idxc, pa, qa, pb, qb, ga, gb, sra, sca, srb, scb, swb):
        w = lax.axis_index("s") * NC + lax.axis_index("c")
        pltpu.sync_copy(row_hbm.at[w], idxr)
        pltpu.sync_copy(col_hbm.at[w], idxc)

        himask = jnp.full((16,), -65536, jnp.int32)

        def add_rows(pref, qref, gref):
            def rbody(r, carry):
                for j in range(dw // 16):
                    sl = pl.ds(j * 16, 16)
                    pw = pref[r, sl]
                    qw = qref[r, sl]
                    lo = (plsc.bitcast(pw << 16, jnp.float32)
                          + plsc.bitcast(qw << 16, jnp.float32))
                    hi = (plsc.bitcast(pw & himask, jnp.float32)
                          + plsc.bitcast(qw & himask, jnp.float32))
                    gref[r, sl] = lo
                    gref[r, pl.ds(dw + j * 16, 16)] = hi
                return carry

            lax.fori_loop(0, chunk, rbody, 0)

        def do_chunk(i, pref, qref, semp, semq):
            cpp = pltpu.async_copy(p_hbm.at[idxr.at[i]], pref, semp)
            cpq = pltpu.async_copy(q_hbm.at[idxc.at[i]], qref, semq)
            return cpp, cpq

        def body(jj, carry):
            i0 = 2 * jj
            i1 = i0 + 1
            cpa, cqa = do_chunk(i0, pa, qa, sra, sca)
            cpb, cqb = do_chunk(i1, pb, qb, srb, scb)
            cpa.wait()
            cqa.wait()
            add_rows(pa, qa, ga)
            wba = pltpu.async_copy(
                ga, g_hbm.at[pl.ds(w * epw + i0 * chunk, chunk)], swb
            )
            cpb.wait()
            cqb.wait()
            add_rows(pb, qb, gb)
            wba.wait()
            pltpu.sync_copy(gb, g_hbm.at[pl.ds(w * epw + i1 * chunk, chunk)])
            return carry

        lax.fori_loop(0, npair, body, 0)

        if nchunk % 2:
            it = nchunk - 1
            cpa, cqa = do_chunk(it, pa, qa, sra, sca)
            cpa.wait()
            cqa.wait()
            add_rows(pa, qa, ga)
            pltpu.sync_copy(ga, g_hbm.at[pl.ds(w * epw + it * chunk, chunk)])

    return k(row3, col3, p32, q32)


def _gather_sum(row3, col3, p, q, d, ne=N_EDGES, chunk=CHUNK):
    dtype = jnp.float32
    epw = ne // NW
    nchunk = epw // chunk
    npair = nchunk // 2

    @functools.partial(
        pl.kernel,
        out_type=jax.ShapeDtypeStruct((ne, d), dtype),
        mesh=_sc_mesh(),
        scratch_types=[
            pltpu.VMEM((nchunk, chunk), jnp.int32),
            pltpu.VMEM((nchunk, chunk), jnp.int32),
            pltpu.VMEM((chunk, d), dtype),
            pltpu.VMEM((chunk, d), dtype),
            pltpu.VMEM((chunk, d), dtype),
            pltpu.VMEM((chunk, d), dtype),
            pltpu.SemaphoreType.DMA,
            pltpu.SemaphoreType.DMA,
            pltpu.SemaphoreType.DMA,
            pltpu.SemaphoreType.DMA,
            pltpu.SemaphoreType.DMA,
        ],
        compiler_params=pltpu.CompilerParams(
            use_tc_tiling_on_sc=(d % 128 == 0)
        ),
    )
    def k(row_hbm, col_hbm, p_hbm, q_hbm, g_hbm,
          idxr, idxc, pa, qa, pb, qb, sra, sca, srb, scb, swb):
        w = lax.axis_index("s") * NC + lax.axis_index("c")
        pltpu.sync_copy(row_hbm.at[w], idxr)
        pltpu.sync_copy(col_hbm.at[w], idxc)

        def add_rows(pref, qref):
            @plsc.parallel_loop(0, chunk, unroll=4)
            def rbody(r):
                for j in range(d // 16):
                    sl = pl.ds(j * 16, 16)
                    pref[r, sl] = pref[r, sl] + qref[r, sl]

        def do_chunk(i, pref, qref, semp, semq):
            cpp = pltpu.async_copy(p_hbm.at[idxr.at[i]], pref, semp)
            cpq = pltpu.async_copy(q_hbm.at[idxc.at[i]], qref, semq)
            return cpp, cpq

        def body(jj, carry):
            i0 = 2 * jj
            i1 = i0 + 1
            cpa, cqa = do_chunk(i0, pa, qa, sra, sca)
            cpb, cqb = do_chunk(i1, pb, qb, srb, scb)
            cpa.wait()
            cqa.wait()
            add_rows(pa, qa)
            wba = pltpu.async_copy(
                pa, g_hbm.at[pl.ds(w * epw + i0 * chunk, chunk)], swb
            )
            cpb.wait()
            cqb.wait()
            add_rows(pb, qb)
            wba.wait()
            pltpu.sync_copy(pb, g_hbm.at[pl.ds(w * epw + i1 * chunk, chunk)])
            return carry

        lax.fori_loop(0, npair, body, 0)

        if nchunk % 2:
            it = nchunk - 1
            cpa, cqa = do_chunk(it, pa, qa, sra, sca)
            cpa.wait()
            cqa.wait()
            add_rows(pa, qa)
            pltpu.sync_copy(pa, g_hbm.at[pl.ds(w * epw + it * chunk, chunk)])

    return k(row3, col3, p, q)


def _scatter_add(col3, m2, ne=N_EDGES, chunk=CHUNK):
    epw = ne // NW
    nchunk = epw // chunk
    npair = nchunk // 2

    @functools.partial(
        pl.kernel,
        out_type=jax.ShapeDtypeStruct((NC * N_NODES, OUT_C), jnp.float32),
        mesh=_sc_mesh(),
        scratch_types=[
            pltpu.VMEM((nchunk, chunk), jnp.int32),
            pltpu.VMEM((chunk, OUT_C // 2), jnp.int32),
            pltpu.VMEM((chunk, OUT_C // 2), jnp.int32),
            pltpu.VMEM((chunk, OUT_C), jnp.float32),
            pltpu.VMEM((chunk, OUT_C), jnp.float32),
            pltpu.VMEM_SHARED((N_NODES, OUT_C), jnp.float32),
            pltpu.SemaphoreType.DMA,
            pltpu.SemaphoreType.DMA,
            pltpu.SemaphoreType.DMA,
            pltpu.SemaphoreType.DMA,
        ],
        compiler_params=pltpu.CompilerParams(
            use_tc_tiling_on_sc=False, needs_layout_passes=False
        ),
    )
    def k(col_hbm, m2_hbm, out_hbm, idxc, pka, pkb, bufa, bufb, acc,
          sla, slb, ssa, ssb):
        dw = OUT_C // 2
        himask = jnp.full((16,), -65536, jnp.int32)

        def unpack_rows(pkref, fref):
            def ubody(r, carry):
                for j in range(dw // 16):
                    sl = pl.ds(j * 16, 16)
                    wv = pkref[r, sl]
                    fref[r, sl] = plsc.bitcast(wv << 16, jnp.float32)
                    fref[r, pl.ds(dw + j * 16, 16)] = plsc.bitcast(
                        wv & himask, jnp.float32
                    )
                return carry

            lax.fori_loop(0, chunk, ubody, 0)
        cid = lax.axis_index("c")
        sid = lax.axis_index("s")
        w = sid * NC + cid
        pltpu.sync_copy(col_hbm.at[w], idxc)

        zvec = plsc.bitcast(himask ^ himask, jnp.float32)

        def zb(kk, carry):
            i = kk // (OUT_C // 16)
            j = kk % (OUT_C // 16)
            bufa[i, pl.ds(j * 16, 16)] = zvec
            return carry

        lax.fori_loop(0, chunk * (OUT_C // 16), zb, 0)

        def zc(kk, carry):
            pltpu.sync_copy(
                bufa, acc.at[pl.ds(sid * TILE_ROWS + kk * chunk, chunk)]
            )
            return carry

        lax.fori_loop(0, TILE_ROWS // chunk, zc, 0)
        if TILE_ROWS % chunk:
            pltpu.sync_copy(
                bufa.at[pl.ds(0, TILE_ROWS % chunk)],
                acc.at[pl.ds(sid * TILE_ROWS + (TILE_ROWS // chunk) * chunk,
                             TILE_ROWS % chunk)],
            )

        @pl.when(sid == NS - 1)
        def _zero_tail():
            pltpu.sync_copy(
                bufa.at[pl.ds(0, TAIL_ROWS)],
                acc.at[pl.ds(NS * TILE_ROWS, TAIL_ROWS)],
            )

        plsc.subcore_barrier()

        def body(jj, carry):
            i0 = 2 * jj
            i1 = i0 + 1
            la = pltpu.async_copy(
                m2_hbm.at[pl.ds(w * epw + i0 * chunk, chunk)], pka, sla
            )
            lb = pltpu.async_copy(
                m2_hbm.at[pl.ds(w * epw + i1 * chunk, chunk)], pkb, slb
            )
            la.wait()
            unpack_rows(pka, bufa)
            sa = pltpu.async_copy(bufa, acc.at[idxc.at[i0]], ssa, add=True)
            lb.wait()
            unpack_rows(pkb, bufb)
            sb = pltpu.async_copy(bufb, acc.at[idxc.at[i1]], ssb, add=True)
            sa.wait()
            sb.wait()
            return carry

        lax.fori_loop(0, npair, body, 0)

        if nchunk % 2:
            it = nchunk - 1
            pltpu.sync_copy(m2_hbm.at[pl.ds(w * epw + it * chunk, chunk)], pka)
            unpack_rows(pka, bufa)
            pltpu.sync_copy(bufa, acc.at[idxc.at[it]], add=True)

        plsc.subcore_barrier()

        r = sid * TILE_ROWS
        pltpu.sync_copy(
            acc.at[pl.ds(r, TILE_ROWS)],
            out_hbm.at[pl.ds(cid * N_NODES + r, TILE_ROWS)],
        )

        @pl.when(sid == NS - 1)
        def _write_tail():
            pltpu.sync_copy(
                acc.at[pl.ds(NS * TILE_ROWS, TAIL_ROWS)],
                out_hbm.at[pl.ds(cid * N_NODES + NS * TILE_ROWS, TAIL_ROWS)],
            )

    return k(col3, m2)


def _precompute_tables(x, wcat):
    def body(x_ref, w_ref, p_ref, q_ref, xr_ref):
        o = jnp.dot(x_ref[...], w_ref[...], preferred_element_type=jnp.float32)
        p_ref[...] = o[:, :OUT_C]
        q_ref[...] = o[:, OUT_C:2 * OUT_C]
        xr_ref[...] = o[:, 2 * OUT_C:]

    blk = 2000
    sd = jax.ShapeDtypeStruct((N_NODES, OUT_C), jnp.float32)
    return pl.pallas_call(
        body,
        grid=(N_NODES // blk,),
        in_specs=[
            pl.BlockSpec((blk, IN_C), lambda i: (i, 0)),
            pl.BlockSpec((IN_C, 3 * OUT_C), lambda i: (0, 0)),
        ],
        out_specs=[
            pl.BlockSpec((blk, OUT_C), lambda i: (i, 0)),
            pl.BlockSpec((blk, OUT_C), lambda i: (i, 0)),
            pl.BlockSpec((blk, OUT_C), lambda i: (i, 0)),
        ],
        out_shape=[sd, sd, jax.ShapeDtypeStruct((N_NODES, OUT_C), jnp.float32)],
    )(x, wcat)


def _edge_mlp1(g, ea, w1c, b1, w2, b2, ne=N_EDGES, off=0):
    def body(g_ref, ea_ref, w1c_ref, b1_ref, w2_ref, b2_ref, o_ref):
        m = (
            g_ref[...].astype(jnp.float32)
            + jnp.dot(ea_ref[...], w1c_ref[...], preferred_element_type=jnp.float32)
            + b1_ref[...]
        )
        m = jnp.maximum(m, 0.0)
        m = jnp.dot(m, w2_ref[...], preferred_element_type=jnp.float32) + b2_ref[...]
        m = jnp.maximum(m, 0.0)
        lo = lax.bitcast_convert_type(
            m[:, :OUT_C // 2].astype(jnp.bfloat16), jnp.uint16
        ).astype(jnp.uint32)
        hi = lax.bitcast_convert_type(
            m[:, OUT_C // 2:].astype(jnp.bfloat16), jnp.uint16
        ).astype(jnp.uint32)
        o_ref[...] = lax.bitcast_convert_type(lo | (hi << 16), jnp.int32)

    blk = 6400
    offb = off // blk
    return pl.pallas_call(
        body,
        grid=(ne // blk,),
        in_specs=[
            pl.BlockSpec((blk, OUT_C), lambda i: (i, 0)),
            pl.BlockSpec((blk, EIN), lambda i, o=offb: (i + o, 0)),
            pl.BlockSpec((EIN, OUT_C), lambda i: (0, 0)),
            pl.BlockSpec((1, OUT_C), lambda i: (0, 0)),
            pl.BlockSpec((OUT_C, OUT_C), lambda i: (0, 0)),
            pl.BlockSpec((1, OUT_C), lambda i: (0, 0)),
        ],
        out_specs=pl.BlockSpec((blk, OUT_C // 2), lambda i: (i, 0)),
        out_shape=jax.ShapeDtypeStruct((ne, OUT_C // 2), jnp.int32),
    )(g, ea, w1c, b1, w2, b2)


def _node_update(aggr_a, aggr_b, xr, gamma, beta, w3a, w3b):
    def body(a0_ref, a1_ref, xr_ref, g_ref, b_ref, wa_ref, wb_ref,
             h_ref, ha_ref, hb_ref):
        h = (a0_ref[:N_NODES, :] + a0_ref[N_NODES:, :]
             + a1_ref[:N_NODES, :] + a1_ref[N_NODES:, :] + xr_ref[...])
        mean = jnp.mean(h, axis=0, keepdims=True)
        var = jnp.mean((h - mean) ** 2, axis=0, keepdims=True)
        h = g_ref[...] * (h - mean) / jnp.sqrt(var + EPS) + b_ref[...]
        h = jnp.maximum(h, 0.0)
        h_ref[...] = h
        ha_ref[...] = jnp.dot(h, wa_ref[...], preferred_element_type=jnp.float32)
        hb_ref[...] = jnp.dot(h, wb_ref[...], preferred_element_type=jnp.float32)

    return pl.pallas_call(
        body,
        out_shape=[
            jax.ShapeDtypeStruct((N_NODES, OUT_C), jnp.float32),
            jax.ShapeDtypeStruct((N_NODES, EOUT), jnp.float32),
            jax.ShapeDtypeStruct((N_NODES, EOUT), jnp.float32),
        ],
    )(aggr_a, aggr_b, xr, gamma, beta, w3a, w3b)


def _edge_mlp2(t, ea, w3c, b3, w4, b4):
    def body(t_ref, ea_ref, w3c_ref, b3_ref, w4_ref, b4_ref, o_ref):
        a = (
            t_ref[...]
            + jnp.dot(ea_ref[...], w3c_ref[...], preferred_element_type=jnp.float32)
            + b3_ref[...]
        )
        a = jnp.maximum(a, 0.0)
        a = jnp.dot(a, w4_ref[...], preferred_element_type=jnp.float32) + b4_ref[...]
        o_ref[...] = jnp.maximum(a, 0.0)

    blk = 12800
    return pl.pallas_call(
        body,
        grid=(N_EDGES // blk,),
        in_specs=[
            pl.BlockSpec((blk, EOUT), lambda i: (i, 0)),
            pl.BlockSpec((blk, EIN), lambda i: (i, 0)),
            pl.BlockSpec((EIN, EOUT), lambda i: (0, 0)),
            pl.BlockSpec((1, EOUT), lambda i: (0, 0)),
            pl.BlockSpec((EOUT, EOUT), lambda i: (0, 0)),
            pl.BlockSpec((1, EOUT), lambda i: (0, 0)),
        ],
        out_specs=pl.BlockSpec((blk, EOUT), lambda i: (i, 0)),
        out_shape=jax.ShapeDtypeStruct((N_EDGES, EOUT), jnp.float32),
    )(t, ea, w3c, b3, w4, b4)


def kernel(x, edge_index, edge_attr, W1, b1, W2, b2, W_root, gamma, beta,
           W3, b3, W4, b4):
    row = edge_index[0]
    col = edge_index[1]

    eh = N_EDGES // 2
    ch = 40
    nh = eh // NW // ch
    rowa = row[:eh].reshape(NW, nh, ch)
    cola = col[:eh].reshape(NW, nh, ch)
    rowb = row[eh:].reshape(NW, nh, ch)
    colb = col[eh:].reshape(NW, nh, ch)

    wcat = jnp.concatenate([W1[:IN_C], W1[IN_C:2 * IN_C], W_root], axis=1)
    p, q, xr = _precompute_tables(x, wcat)

    ga = _gather_sum(rowa, cola, p, q, OUT_C, ne=eh, chunk=ch)
    gb = _gather_sum(rowb, colb, p, q, OUT_C, ne=eh, chunk=ch)

    w1c = W1[2 * IN_C:]
    b1r = b1.reshape(1, OUT_C)
    b2r = b2.reshape(1, OUT_C)
    m2a = _edge_mlp1(ga, edge_attr, w1c, b1r, W2, b2r, ne=eh, off=0)
    m2b = _edge_mlp1(gb, edge_attr, w1c, b1r, W2, b2r, ne=eh, off=eh)

    aggr_a = _scatter_add(cola, m2a, ne=eh, chunk=ch)
    aggr_b = _scatter_add(colb, m2b, ne=eh, chunk=ch)

    h, ha, hb = _node_update(
        aggr_a, aggr_b, xr,
        gamma.reshape(1, OUT_C), beta.reshape(1, OUT_C),
        W3[:OUT_C], W3[OUT_C:2 * OUT_C],
    )

    row3 = row.reshape(NW, NCHUNK, CHUNK)
    col3 = col.reshape(NW, NCHUNK, CHUNK)

    t = _gather_sum(row3, col3, ha, hb, EOUT)

    e = _edge_mlp2(
        t, edge_attr, W3[2 * OUT_C:], b3.reshape(1, EOUT), W4,
        b4.reshape(1, EOUT),
    )
    return (h, edge_index, e)


# speedup vs baseline: 1.1212x; 1.1212x over previous
import functools

import jax
import jax.numpy as jnp
from jax import lax
from jax.experimental import pallas as pl
from jax.experimental.pallas import tpu as pltpu
from jax.experimental.pallas import tpu_sc as plsc

N_NODES = 10000
N_EDGES = 320000
IN_C = 128
OUT_C = 128
EIN = 16
EOUT = 16
EPS = 1e-5

NC = 2
NS = 16
NW = NC * NS
EPW = N_EDGES // NW
CHUNK = 80
NCHUNK = EPW // CHUNK
TILE_ROWS = 624
TAIL_ROWS = N_NODES - NS * TILE_ROWS


def _sc_mesh():
    return plsc.VectorSubcoreMesh(
        core_axis_name="c", subcore_axis_name="s", num_cores=NC, num_subcores=NS
    )


def _gather_sum(row3, col3, p, q, d, ne=N_EDGES, chunk=CHUNK):
    dtype = jnp.float32
    epw = ne // NW
    nchunk = epw // chunk
    npair = nchunk // 2

    @functools.partial(
        pl.kernel,
        out_type=jax.ShapeDtypeStruct((ne, d), dtype),
        mesh=_sc_mesh(),
        scratch_types=[
            pltpu.VMEM((nchunk, chunk), jnp.int32),
            pltpu.VMEM((nchunk, chunk), jnp.int32),
            pltpu.VMEM((chunk, d), dtype),
            pltpu.VMEM((chunk, d), dtype),
            pltpu.VMEM((chunk, d), dtype),
            pltpu.VMEM((chunk, d), dtype),
            pltpu.SemaphoreType.DMA,
            pltpu.SemaphoreType.DMA,
            pltpu.SemaphoreType.DMA,
            pltpu.SemaphoreType.DMA,
            pltpu.SemaphoreType.DMA,
        ],
        compiler_params=pltpu.CompilerParams(
            use_tc_tiling_on_sc=(d % 128 == 0)
        ),
    )
    def k(row_hbm, col_hbm, p_hbm, q_hbm, g_hbm,
          idxr, idxc, pa, qa, pb, qb, sra, sca, srb, scb, swb):
        w = lax.axis_index("s") * NC + lax.axis_index("c")
        pltpu.sync_copy(row_hbm.at[w], idxr)
        pltpu.sync_copy(col_hbm.at[w], idxc)

        def add_rows(pref, qref):
            @plsc.parallel_loop(0, chunk, unroll=4)
            def rbody(r):
                for j in range(d // 16):
                    sl = pl.ds(j * 16, 16)
                    pref[r, sl] = pref[r, sl] + qref[r, sl]

        def do_chunk(i, pref, qref, semp, semq):
            cpp = pltpu.async_copy(p_hbm.at[idxr.at[i]], pref, semp)
            cpq = pltpu.async_copy(q_hbm.at[idxc.at[i]], qref, semq)
            return cpp, cpq

        def body(jj, carry):
            i0 = 2 * jj
            i1 = i0 + 1
            cpa, cqa = do_chunk(i0, pa, qa, sra, sca)
            cpb, cqb = do_chunk(i1, pb, qb, srb, scb)
            cpa.wait()
            cqa.wait()
            add_rows(pa, qa)
            wba = pltpu.async_copy(
                pa, g_hbm.at[pl.ds(w * epw + i0 * chunk, chunk)], swb
            )
            cpb.wait()
            cqb.wait()
            add_rows(pb, qb)
            wba.wait()
            pltpu.sync_copy(pb, g_hbm.at[pl.ds(w * epw + i1 * chunk, chunk)])
            return carry

        lax.fori_loop(0, npair, body, 0)

        if nchunk % 2:
            it = nchunk - 1
            cpa, cqa = do_chunk(it, pa, qa, sra, sca)
            cpa.wait()
            cqa.wait()
            add_rows(pa, qa)
            pltpu.sync_copy(pa, g_hbm.at[pl.ds(w * epw + it * chunk, chunk)])

    return k(row3, col3, p, q)


def _scatter_add(col3, m2, ne=N_EDGES, chunk=CHUNK):
    epw = ne // NW
    nchunk = epw // chunk
    npair = nchunk // 2

    @functools.partial(
        pl.kernel,
        out_type=jax.ShapeDtypeStruct((NC * N_NODES, OUT_C), jnp.float32),
        mesh=_sc_mesh(),
        scratch_types=[
            pltpu.VMEM((nchunk, chunk), jnp.int32),
            pltpu.VMEM((chunk, OUT_C), jnp.float32),
            pltpu.VMEM((chunk, OUT_C), jnp.float32),
            pltpu.VMEM_SHARED((N_NODES, OUT_C), jnp.float32),
            pltpu.SemaphoreType.DMA,
            pltpu.SemaphoreType.DMA,
            pltpu.SemaphoreType.DMA,
            pltpu.SemaphoreType.DMA,
        ],
    )
    def k(col_hbm, m2_hbm, out_hbm, idxc, bufa, bufb, acc, sla, slb, ssa, ssb):
        cid = lax.axis_index("c")
        sid = lax.axis_index("s")
        w = sid * NC + cid
        pltpu.sync_copy(col_hbm.at[w], idxc)

        def zb(kk, carry):
            i = kk // (OUT_C // 16)
            j = kk % (OUT_C // 16)
            bufa[i, pl.ds(j * 16, 16)] = jnp.zeros((16,), jnp.float32)
            return carry

        lax.fori_loop(0, chunk * (OUT_C // 16), zb, 0)

        def zc(kk, carry):
            pltpu.sync_copy(
                bufa, acc.at[pl.ds(sid * TILE_ROWS + kk * chunk, chunk)]
            )
            return carry

        lax.fori_loop(0, TILE_ROWS // chunk, zc, 0)
        if TILE_ROWS % chunk:
            pltpu.sync_copy(
                bufa.at[pl.ds(0, TILE_ROWS % chunk)],
                acc.at[pl.ds(sid * TILE_ROWS + (TILE_ROWS // chunk) * chunk,
                             TILE_ROWS % chunk)],
            )

        @pl.when(sid == NS - 1)
        def _zero_tail():
            pltpu.sync_copy(
                bufa.at[pl.ds(0, TAIL_ROWS)],
                acc.at[pl.ds(NS * TILE_ROWS, TAIL_ROWS)],
            )

        plsc.subcore_barrier()

        def body(jj, carry):
            i0 = 2 * jj
            i1 = i0 + 1
            la = pltpu.async_copy(
                m2_hbm.at[pl.ds(w * epw + i0 * chunk, chunk)], bufa, sla
            )
            lb = pltpu.async_copy(
                m2_hbm.at[pl.ds(w * epw + i1 * chunk, chunk)], bufb, slb
            )
            la.wait()
            sa = pltpu.async_copy(bufa, acc.at[idxc.at[i0]], ssa, add=True)
            lb.wait()
            sb = pltpu.async_copy(bufb, acc.at[idxc.at[i1]], ssb, add=True)
            sa.wait()
            sb.wait()
            return carry

        lax.fori_loop(0, npair, body, 0)

        if nchunk % 2:
            it = nchunk - 1
            pltpu.sync_copy(m2_hbm.at[pl.ds(w * epw + it * chunk, chunk)], bufa)
            pltpu.sync_copy(bufa, acc.at[idxc.at[it]], add=True)

        plsc.subcore_barrier()

        r = sid * TILE_ROWS
        pltpu.sync_copy(
            acc.at[pl.ds(r, TILE_ROWS)],
            out_hbm.at[pl.ds(cid * N_NODES + r, TILE_ROWS)],
        )

        @pl.when(sid == NS - 1)
        def _write_tail():
            pltpu.sync_copy(
                acc.at[pl.ds(NS * TILE_ROWS, TAIL_ROWS)],
                out_hbm.at[pl.ds(cid * N_NODES + NS * TILE_ROWS, TAIL_ROWS)],
            )

    return k(col3, m2)


def _precompute_tables(x, wcat):
    def body(x_ref, w_ref, p_ref, q_ref, xr_ref):
        o = jnp.dot(x_ref[...], w_ref[...], preferred_element_type=jnp.float32)
        p_ref[...] = o[:, :OUT_C]
        q_ref[...] = o[:, OUT_C:2 * OUT_C]
        xr_ref[...] = o[:, 2 * OUT_C:]

    blk = 2000
    sd = jax.ShapeDtypeStruct((N_NODES, OUT_C), jnp.float32)
    return pl.pallas_call(
        body,
        grid=(N_NODES // blk,),
        in_specs=[
            pl.BlockSpec((blk, IN_C), lambda i: (i, 0)),
            pl.BlockSpec((IN_C, 3 * OUT_C), lambda i: (0, 0)),
        ],
        out_specs=[
            pl.BlockSpec((blk, OUT_C), lambda i: (i, 0)),
            pl.BlockSpec((blk, OUT_C), lambda i: (i, 0)),
            pl.BlockSpec((blk, OUT_C), lambda i: (i, 0)),
        ],
        out_shape=[sd, sd, jax.ShapeDtypeStruct((N_NODES, OUT_C), jnp.float32)],
    )(x, wcat)


def _edge_mlp1(g, ea, w1c, b1, w2, b2, ne=N_EDGES, off=0):
    def body(g_ref, ea_ref, w1c_ref, b1_ref, w2_ref, b2_ref, o_ref):
        m = (
            g_ref[...].astype(jnp.float32)
            + jnp.dot(ea_ref[...], w1c_ref[...], preferred_element_type=jnp.float32)
            + b1_ref[...]
        )
        m = jnp.maximum(m, 0.0)
        m = jnp.dot(m, w2_ref[...], preferred_element_type=jnp.float32) + b2_ref[...]
        o_ref[...] = jnp.maximum(m, 0.0)

    blk = 6400
    offb = off // blk
    return pl.pallas_call(
        body,
        grid=(ne // blk,),
        in_specs=[
            pl.BlockSpec((blk, OUT_C), lambda i: (i, 0)),
            pl.BlockSpec((blk, EIN), lambda i, o=offb: (i + o, 0)),
            pl.BlockSpec((EIN, OUT_C), lambda i: (0, 0)),
            pl.BlockSpec((1, OUT_C), lambda i: (0, 0)),
            pl.BlockSpec((OUT_C, OUT_C), lambda i: (0, 0)),
            pl.BlockSpec((1, OUT_C), lambda i: (0, 0)),
        ],
        out_specs=pl.BlockSpec((blk, OUT_C), lambda i: (i, 0)),
        out_shape=jax.ShapeDtypeStruct((ne, OUT_C), jnp.float32),
    )(g, ea, w1c, b1, w2, b2)


def _node_update(aggr_a, aggr_b, xr, gamma, beta, w3a, w3b):
    def body(a0_ref, a1_ref, xr_ref, g_ref, b_ref, wa_ref, wb_ref,
             h_ref, ha_ref, hb_ref):
        h = (a0_ref[:N_NODES, :] + a0_ref[N_NODES:, :]
             + a1_ref[:N_NODES, :] + a1_ref[N_NODES:, :] + xr_ref[...])
        mean = jnp.mean(h, axis=0, keepdims=True)
        var = jnp.mean((h - mean) ** 2, axis=0, keepdims=True)
        h = g_ref[...] * (h - mean) / jnp.sqrt(var + EPS) + b_ref[...]
        h = jnp.maximum(h, 0.0)
        h_ref[...] = h
        ha_ref[...] = jnp.dot(h, wa_ref[...], preferred_element_type=jnp.float32)
        hb_ref[...] = jnp.dot(h, wb_ref[...], preferred_element_type=jnp.float32)

    return pl.pallas_call(
        body,
        out_shape=[
            jax.ShapeDtypeStruct((N_NODES, OUT_C), jnp.float32),
            jax.ShapeDtypeStruct((N_NODES, EOUT), jnp.float32),
            jax.ShapeDtypeStruct((N_NODES, EOUT), jnp.float32),
        ],
    )(aggr_a, aggr_b, xr, gamma, beta, w3a, w3b)


def _edge_mlp2(t, ea, w3c, b3, w4, b4):
    def body(t_ref, ea_ref, w3c_ref, b3_ref, w4_ref, b4_ref, o_ref):
        a = (
            t_ref[...]
            + jnp.dot(ea_ref[...], w3c_ref[...], preferred_element_type=jnp.float32)
            + b3_ref[...]
        )
        a = jnp.maximum(a, 0.0)
        a = jnp.dot(a, w4_ref[...], preferred_element_type=jnp.float32) + b4_ref[...]
        o_ref[...] = jnp.maximum(a, 0.0)

    blk = 12800
    return pl.pallas_call(
        body,
        grid=(N_EDGES // blk,),
        in_specs=[
            pl.BlockSpec((blk, EOUT), lambda i: (i, 0)),
            pl.BlockSpec((blk, EIN), lambda i: (i, 0)),
            pl.BlockSpec((EIN, EOUT), lambda i: (0, 0)),
            pl.BlockSpec((1, EOUT), lambda i: (0, 0)),
            pl.BlockSpec((EOUT, EOUT), lambda i: (0, 0)),
            pl.BlockSpec((1, EOUT), lambda i: (0, 0)),
        ],
        out_specs=pl.BlockSpec((blk, EOUT), lambda i: (i, 0)),
        out_shape=jax.ShapeDtypeStruct((N_EDGES, EOUT), jnp.float32),
    )(t, ea, w3c, b3, w4, b4)


def kernel(x, edge_index, edge_attr, W1, b1, W2, b2, W_root, gamma, beta,
           W3, b3, W4, b4):
    row = edge_index[0]
    col = edge_index[1]

    eh = N_EDGES // 2
    ch = 40
    nh = eh // NW // ch
    rowa = row[:eh].reshape(NW, nh, ch)
    cola = col[:eh].reshape(NW, nh, ch)
    rowb = row[eh:].reshape(NW, nh, ch)
    colb = col[eh:].reshape(NW, nh, ch)

    wcat = jnp.concatenate([W1[:IN_C], W1[IN_C:2 * IN_C], W_root], axis=1)
    p, q, xr = _precompute_tables(x, wcat)

    ga = _gather_sum(rowa, cola, p, q, OUT_C, ne=eh, chunk=ch)
    gb = _gather_sum(rowb, colb, p, q, OUT_C, ne=eh, chunk=ch)

    w1c = W1[2 * IN_C:]
    b1r = b1.reshape(1, OUT_C)
    b2r = b2.reshape(1, OUT_C)
    m2a = _edge_mlp1(ga, edge_attr, w1c, b1r, W2, b2r, ne=eh, off=0)
    m2b = _edge_mlp1(gb, edge_attr, w1c, b1r, W2, b2r, ne=eh, off=eh)

    aggr_a = _scatter_add(cola, m2a, ne=eh, chunk=ch)
    aggr_b = _scatter_add(colb, m2b, ne=eh, chunk=ch)

    h, ha, hb = _node_update(
        aggr_a, aggr_b, xr,
        gamma.reshape(1, OUT_C), beta.reshape(1, OUT_C),
        W3[:OUT_C], W3[OUT_C:2 * OUT_C],
    )

    row3 = row.reshape(NW, NCHUNK, CHUNK)
    col3 = col.reshape(NW, NCHUNK, CHUNK)

    t = _gather_sum(row3, col3, ha, hb, EOUT)

    e = _edge_mlp2(
        t, edge_attr, W3[2 * OUT_C:], b3.reshape(1, EOUT), W4,
        b4.reshape(1, EOUT),
    )
    return (h, edge_index, e)
